# Initial kernel scaffold; baseline (speedup 1.0000x reference)
import jax
import jax.numpy as jnp
from jax.experimental import pallas as pl


def kernel(cloud, indices, W1, b1, g1, e1, W2, b2, g2, e2, W3, b3, g3, e3, W4, b4, g4, e4, Wf, bf, gf, ef, Wm1, bm1, gm1, em1, Wm2, bm2, gm2, em2, Wo, bo):
    raise NotImplementedError("write your pallas kernel here")


# speedup vs baseline: 7.2077x; 7.2077x over previous
import functools

import jax
import jax.numpy as jnp
import numpy as np
from jax import lax
from jax.experimental import pallas as pl
from jax.experimental.pallas import tpu as pltpu
from jax.experimental.pallas import tpu_sc as plsc

B = 16
N = 1024
K = 20
KP = 24
NEG = np.float32(-np.inf)


def _leaky(y):
    return jnp.maximum(y, 0.2 * y)


def _layer1_body(x_ref, wa_ref, wba_ref, bp_ref, u_ref, v_ref):
    x = x_ref[0]
    u_ref[0] = jnp.dot(x, wa_ref[...], preferred_element_type=jnp.float32)
    v_ref[0] = (
        jnp.dot(x, wba_ref[...], preferred_element_type=jnp.float32)
        + bp_ref[...]
    )


def _layer1(xpad, wa, wba, bp, co):
    ci = xpad.shape[-1]
    return pl.pallas_call(
        _layer1_body,
        grid=(B,),
        in_specs=[
            pl.BlockSpec((1, N, ci), lambda b: (b, 0, 0)),
            pl.BlockSpec((ci, co), lambda b: (0, 0)),
            pl.BlockSpec((ci, co), lambda b: (0, 0)),
            pl.BlockSpec((1, co), lambda b: (0, 0)),
        ],
        out_specs=[
            pl.BlockSpec((1, N, co), lambda b: (b, 0, 0)),
            pl.BlockSpec((1, N, co), lambda b: (b, 0, 0)),
        ],
        out_shape=[
            jax.ShapeDtypeStruct((B, N, co), jnp.float32),
            jax.ShapeDtypeStruct((B, N, co), jnp.float32),
        ],
    )(xpad, wa, wba, bp)


def _layer_body(m_ref, v_ref, wa_ref, wba_ref, bp_ref,
                x_ref, idx_ref, u_ref, vo_ref, d_ref):
    m = m_ref[0]
    v = v_ref[0]
    x = _leaky(m + v)
    x_ref[0] = x
    xx = jnp.sum(x * x, axis=1, keepdims=True)
    inner = lax.dot_general(
        x, x, (((1,), (1,)), ((), ())), preferred_element_type=jnp.float32
    )
    d_ref[...] = -((xx - 2.0 * inner) + jnp.reshape(xx, (1, N)))

    lane = lax.broadcasted_iota(jnp.int32, (N, N), 1)
    lane24 = lax.broadcasted_iota(jnp.int32, (N, KP), 1)
    n_i32 = np.int32(N)

    def step(j, carry):
        pv, pi, acc = carry
        d = d_ref[...]
        elig = (d < pv) | ((d == pv) & (lane > pi))
        cand = jnp.where(elig, d, NEG)
        rm = jnp.max(cand, axis=1, keepdims=True)
        msk = elig & (d == rm)
        ii = jnp.min(jnp.where(msk, lane, n_i32), axis=1, keepdims=True)
        acc = jnp.where(lane24 == j, ii, acc)
        return rm, ii, acc

    pv0 = jnp.full((N, 1), jnp.inf, jnp.float32)
    pi0 = jnp.full((N, 1), -1, jnp.int32)
    acc0 = jnp.zeros((N, KP), jnp.int32)
    _, _, acc = lax.fori_loop(0, K, step, (pv0, pi0, acc0))
    acc = jnp.where(lane24 >= K, acc[:, 0:1], acc)
    idx_ref[0] = acc

    u_ref[0] = jnp.dot(x, wa_ref[...], preferred_element_type=jnp.float32)
    vo_ref[0] = (
        jnp.dot(x, wba_ref[...], preferred_element_type=jnp.float32)
        + bp_ref[...]
    )


def _layer(m, v, wa, wba, bp):
    ci, co = wa.shape
    return pl.pallas_call(
        _layer_body,
        grid=(B,),
        in_specs=[
            pl.BlockSpec((1, N, ci), lambda b: (b, 0, 0)),
            pl.BlockSpec((1, N, ci), lambda b: (b, 0, 0)),
            pl.BlockSpec((ci, co), lambda b: (0, 0)),
            pl.BlockSpec((ci, co), lambda b: (0, 0)),
            pl.BlockSpec((1, co), lambda b: (0, 0)),
        ],
        out_specs=[
            pl.BlockSpec((1, N, ci), lambda b: (b, 0, 0)),
            pl.BlockSpec((1, N, KP), lambda b: (b, 0, 0)),
            pl.BlockSpec((1, N, co), lambda b: (b, 0, 0)),
            pl.BlockSpec((1, N, co), lambda b: (b, 0, 0)),
        ],
        out_shape=[
            jax.ShapeDtypeStruct((B, N, ci), jnp.float32),
            jax.ShapeDtypeStruct((B, N, KP), jnp.int32),
            jax.ShapeDtypeStruct((B, N, co), jnp.float32),
            jax.ShapeDtypeStruct((B, N, co), jnp.float32),
        ],
        scratch_shapes=[pltpu.VMEM((N, N), jnp.float32)],
    )(m, v, wa, wba, bp)


def _gather_max(table, idx_flat):
    co = table.shape[-1]
    nw = 32
    rows_per_w = (B * N) // nw
    R = 4
    n_chunks = rows_per_w // R
    ncc = co // 16

    mesh = plsc.VectorSubcoreMesh(core_axis_name="c", subcore_axis_name="s")

    @functools.partial(
        pl.kernel,
        out_type=jax.ShapeDtypeStruct((B * N, co), jnp.float32),
        mesh=mesh,
        scratch_types=[
            pltpu.VMEM((R * KP,), jnp.int32),
            pltpu.VMEM((R * KP,), jnp.int32),
            pltpu.VMEM((R * KP, co), jnp.float32),
            pltpu.VMEM((R, co), jnp.float32),
            pltpu.SemaphoreType.DMA,
        ],
    )
    def k(table_hbm, idx_hbm, out_hbm, idxr_v, gidx_v, rows_v, out_v, sem):
        wid = lax.axis_index("s") * 2 + lax.axis_index("c")
        base_row = wid * rows_per_w
        boff = (wid // 2) * N

        def chunk(i, _):
            r0 = base_row + i * R
            pltpu.sync_copy(idx_hbm.at[pl.ds(r0 * KP, R * KP)], idxr_v)
            for t in range(R * KP // 16):
                gidx_v[pl.ds(t * 16, 16)] = idxr_v[pl.ds(t * 16, 16)] + boff
            pltpu.async_copy(table_hbm.at[gidx_v], rows_v, sem).wait()
            for r in range(R):
                for cc in range(ncc):
                    acc = rows_v[r * KP, pl.ds(cc * 16, 16)]
                    for j in range(1, KP):
                        acc = jnp.maximum(
                            acc, rows_v[r * KP + j, pl.ds(cc * 16, 16)]
                        )
                    out_v[r, pl.ds(cc * 16, 16)] = acc
            pltpu.sync_copy(out_v, out_hbm.at[pl.ds(r0, R)])
            return 0

        lax.fori_loop(0, n_chunks, chunk, 0)

    return k(table, idx_flat)


def _head1_body(m_ref, v_ref, x1_ref, x2_ref, x3_ref, wf_ref, bf_ref, o_ref):
    x4 = _leaky(m_ref[0] + v_ref[0])
    xcat = jnp.concatenate(
        [x1_ref[0][:, :64], x2_ref[0][:, :64], x3_ref[0], x4], axis=1)
    y = _leaky(
        jnp.dot(xcat, wf_ref[...], preferred_element_type=jnp.float32)
        + bf_ref[...]
    )
    mx = jnp.max(y, axis=0, keepdims=True)
    mn = jnp.sum(y, axis=0, keepdims=True) * np.float32(1.0 / N)
    o_ref[0] = jnp.concatenate([mx, mn], axis=1)


def _head1(m4, v4, x1, x2, x3, wf_t, bfp):
    return pl.pallas_call(
        _head1_body,
        grid=(B,),
        in_specs=[
            pl.BlockSpec((1, N, 256), lambda b: (b, 0, 0)),
            pl.BlockSpec((1, N, 256), lambda b: (b, 0, 0)),
            pl.BlockSpec((1, N, 128), lambda b: (b, 0, 0)),
            pl.BlockSpec((1, N, 128), lambda b: (b, 0, 0)),
            pl.BlockSpec((1, N, 128), lambda b: (b, 0, 0)),
            pl.BlockSpec((512, 1024), lambda b: (0, 0)),
            pl.BlockSpec((1, 1024), lambda b: (0, 0)),
        ],
        out_specs=pl.BlockSpec((1, 1, 2048), lambda b: (b, 0, 0)),
        out_shape=jax.ShapeDtypeStruct((B, 1, 2048), jnp.float32),
    )(m4, v4, x1, x2, x3, wf_t, bfp)


def _head2_body(p_ref, w1_ref, b1_ref, w2_ref, b2_ref, wo_ref, bo_ref, o_ref):
    h = _leaky(
        jnp.dot(p_ref[...], w1_ref[...], preferred_element_type=jnp.float32)
        + b1_ref[...]
    )
    h = _leaky(
        jnp.dot(h, w2_ref[...], preferred_element_type=jnp.float32)
        + b2_ref[...]
    )
    o_ref[...] = (
        jnp.dot(h, wo_ref[...], preferred_element_type=jnp.float32)
        + bo_ref[...]
    )


def _head2(pooled, w1t, b1p, w2t, b2p, wot, bop):
    return pl.pallas_call(
        _head2_body,
        in_specs=[pl.BlockSpec(a.shape, lambda: tuple(0 for _ in a.shape))
                  for a in (pooled, w1t, b1p, w2t, b2p, wot, bop)],
        out_specs=pl.BlockSpec((B, 128), lambda: (0, 0)),
        out_shape=jax.ShapeDtypeStruct((B, 128), jnp.float32),
    )(pooled, w1t, b1p, w2t, b2p, wot, bop)


def _fold_conv(W, bvec, g, e, ci_pad, co_pad):
    Wp = W * g[:, None]
    bp = bvec * g + e
    C = W.shape[1] // 2
    co = W.shape[0]
    Wa = Wp[:, :C].T
    Wba = (Wp[:, C:] - Wp[:, :C]).T

    def pad2(a):
        return jnp.pad(a, ((0, ci_pad - C), (0, co_pad - co)))

    return pad2(Wa), pad2(Wba), jnp.pad(bp, (0, co_pad - co))[None, :]


def kernel(cloud, indices, W1, b1, g1, e1, W2, b2, g2, e2, W3, b3, g3, e3,
           W4, b4, g4, e4, Wf, bf, gf, ef, Wm1, bm1, gm1, em1,
           Wm2, bm2, gm2, em2, Wo, bo):
    wa1, wba1, bp1 = _fold_conv(W1, b1, g1, e1, 8, 128)
    wa2, wba2, bp2 = _fold_conv(W2, b2, g2, e2, 128, 128)
    wa3, wba3, bp3 = _fold_conv(W3, b3, g3, e3, 128, 128)
    wa4, wba4, bp4 = _fold_conv(W4, b4, g4, e4, 128, 256)
    xpad = jnp.concatenate(
        [cloud, jnp.zeros((B, N, 5), jnp.float32)], axis=-1)

    idx1 = indices.astype(jnp.int32)
    idx1 = jnp.concatenate([idx1, idx1[..., :KP - K]], axis=-1)

    u1, v1 = _layer1(xpad, wa1, wba1, bp1, 128)
    m1 = _gather_max(u1.reshape(B * N, 128), idx1.reshape(-1))
    m1 = m1.reshape(B, N, 128)

    x1, idx2, u2, v2 = _layer(m1, v1, wa2, wba2, bp2)
    m2 = _gather_max(u2.reshape(B * N, 128), idx2.reshape(-1)).reshape(B, N, 128)

    x2, idx3, u3, v3 = _layer(m2, v2, wa3, wba3, bp3)
    m3 = _gather_max(u3.reshape(B * N, 128), idx3.reshape(-1)).reshape(B, N, 128)

    x3, idx4, u4, v4 = _layer(m3, v3, wa4, wba4, bp4)
    m4 = _gather_max(u4.reshape(B * N, 256), idx4.reshape(-1)).reshape(B, N, 256)

    wf_t = (Wf * gf[:, None]).T
    bfp = (bf * gf + ef)[None, :]
    pooled = _head1(m4, v4, x1, x2, x3, wf_t, bfp).reshape(B, 2048)

    w1t = (Wm1 * gm1[:, None]).T
    b1p = (bm1 * gm1 + em1)[None, :]
    w2t = (Wm2 * gm2[:, None]).T
    b2p = (bm2 * gm2 + em2)[None, :]
    wot = jnp.concatenate([Wo.T, jnp.zeros((256, 88), jnp.float32)], axis=1)
    bop = jnp.concatenate([bo, jnp.zeros((88,), jnp.float32)])[None, :]
    out = _head2(pooled, w1t, b1p, w2t, b2p, wot, bop)
    return out[:, :40]


# speedup vs baseline: 8.2608x; 1.1461x over previous
import functools

import jax
import jax.numpy as jnp
import numpy as np
from jax import lax
from jax.experimental import pallas as pl
from jax.experimental.pallas import tpu as pltpu
from jax.experimental.pallas import tpu_sc as plsc

B = 16
N = 1024
K = 20
KP = 20
NEG = np.float32(-np.inf)


def _leaky(y):
    return jnp.maximum(y, 0.2 * y)


def _layer1_body(x_ref, wa_ref, wba_ref, bp_ref, u_ref, v_ref):
    x = x_ref[0]
    u_ref[0] = jnp.dot(x, wa_ref[...], preferred_element_type=jnp.float32)
    v_ref[0] = (
        jnp.dot(x, wba_ref[...], preferred_element_type=jnp.float32)
        + bp_ref[...]
    )


def _layer1(xpad, wa, wba, bp, co):
    ci = xpad.shape[-1]
    return pl.pallas_call(
        _layer1_body,
        grid=(B,),
        in_specs=[
            pl.BlockSpec((1, N, ci), lambda b: (b, 0, 0)),
            pl.BlockSpec((ci, co), lambda b: (0, 0)),
            pl.BlockSpec((ci, co), lambda b: (0, 0)),
            pl.BlockSpec((1, co), lambda b: (0, 0)),
        ],
        out_specs=[
            pl.BlockSpec((1, N, co), lambda b: (b, 0, 0)),
            pl.BlockSpec((1, N, co), lambda b: (b, 0, 0)),
        ],
        out_shape=[
            jax.ShapeDtypeStruct((B, N, co), jnp.float32),
            jax.ShapeDtypeStruct((B, N, co), jnp.float32),
        ],
    )(xpad, wa, wba, bp)


def _layer_body(m_ref, v_ref, wa_ref, wba_ref, bp_ref,
                x_ref, idx_ref, u_ref, vo_ref, d_ref):
    m = m_ref[0]
    v = v_ref[0]
    x = _leaky(m + v)
    x_ref[0] = x
    xx = jnp.sum(x * x, axis=1, keepdims=True)
    inner = lax.dot_general(
        x, x, (((1,), (1,)), ((), ())), preferred_element_type=jnp.float32
    )
    d_ref[...] = -((xx - 2.0 * inner) + jnp.reshape(xx, (1, N)))

    lane = lax.broadcasted_iota(jnp.int32, (N, N), 1)
    lane24 = lax.broadcasted_iota(jnp.int32, (N, KP), 1)
    n_i32 = np.int32(N)

    def step(j, carry):
        pv, pi, acc = carry
        d = d_ref[...]
        elig = (d < pv) | ((d == pv) & (lane > pi))
        rm = jnp.max(jnp.where(elig, d, NEG), axis=1, keepdims=True)
        pi_eff = jnp.where(rm == pv, pi, -1)
        msk = (d == rm) & (lane > pi_eff)
        ii = jnp.min(jnp.where(msk, lane, n_i32), axis=1, keepdims=True)
        acc = jnp.where(lane24 == j, ii, acc)
        return rm, ii, acc

    pv0 = jnp.full((N, 1), jnp.inf, jnp.float32)
    pi0 = jnp.full((N, 1), -1, jnp.int32)
    acc0 = jnp.zeros((N, KP), jnp.int32)
    _, _, acc = lax.fori_loop(0, K, step, (pv0, pi0, acc0))
    idx_ref[0] = acc

    u_ref[0] = jnp.dot(x, wa_ref[...], preferred_element_type=jnp.float32)
    vo_ref[0] = (
        jnp.dot(x, wba_ref[...], preferred_element_type=jnp.float32)
        + bp_ref[...]
    )


def _layer(m, v, wa, wba, bp):
    ci, co = wa.shape
    return pl.pallas_call(
        _layer_body,
        grid=(B,),
        in_specs=[
            pl.BlockSpec((1, N, ci), lambda b: (b, 0, 0)),
            pl.BlockSpec((1, N, ci), lambda b: (b, 0, 0)),
            pl.BlockSpec((ci, co), lambda b: (0, 0)),
            pl.BlockSpec((ci, co), lambda b: (0, 0)),
            pl.BlockSpec((1, co), lambda b: (0, 0)),
        ],
        out_specs=[
            pl.BlockSpec((1, N, ci), lambda b: (b, 0, 0)),
            pl.BlockSpec((1, N, KP), lambda b: (b, 0, 0)),
            pl.BlockSpec((1, N, co), lambda b: (b, 0, 0)),
            pl.BlockSpec((1, N, co), lambda b: (b, 0, 0)),
        ],
        out_shape=[
            jax.ShapeDtypeStruct((B, N, ci), jnp.float32),
            jax.ShapeDtypeStruct((B, N, KP), jnp.int32),
            jax.ShapeDtypeStruct((B, N, co), jnp.float32),
            jax.ShapeDtypeStruct((B, N, co), jnp.float32),
        ],
        scratch_shapes=[pltpu.VMEM((N, N), jnp.float32)],
    )(m, v, wa, wba, bp)


def _gather_max(table, idx_flat):
    co = table.shape[-1]
    nw = 32
    rows_per_w = (B * N) // nw
    R = 4
    n_chunks = rows_per_w // R
    ncc = co // 16

    mesh = plsc.VectorSubcoreMesh(core_axis_name="c", subcore_axis_name="s")

    @functools.partial(
        pl.kernel,
        out_type=jax.ShapeDtypeStruct((B * N, co), jnp.float32),
        mesh=mesh,
        scratch_types=[
            pltpu.VMEM((2, R * KP), jnp.int32),
            pltpu.VMEM((2, R * KP), jnp.int32),
            pltpu.VMEM((2, R * KP, co), jnp.float32),
            pltpu.VMEM((2, R, co), jnp.float32),
            pltpu.SemaphoreType.DMA,
            pltpu.SemaphoreType.DMA,
            pltpu.SemaphoreType.DMA,
            pltpu.SemaphoreType.DMA,
        ],
    )
    def k(table_hbm, idx_hbm, out_hbm, idxr_v, gidx_v, rows_v, out_v,
          gs0, gs1, os0, os1):
        wid = lax.axis_index("s") * 2 + lax.axis_index("c")
        base_row = wid * rows_per_w
        boff = (wid // 2) * N
        gsems = (gs0, gs1)
        osems = (os0, os1)

        def fire(p, c):
            r0 = base_row + c * R
            pltpu.sync_copy(idx_hbm.at[pl.ds(r0 * KP, R * KP)],
                            idxr_v.at[p])
            for t in range(R * KP // 16):
                gidx_v[p, pl.ds(t * 16, 16)] = (
                    idxr_v[p, pl.ds(t * 16, 16)] + boff)
            pltpu.async_copy(table_hbm.at[gidx_v.at[p]], rows_v.at[p],
                             gsems[p])

        def consume(p, c):
            r0 = base_row + c * R
            pltpu.make_async_copy(table_hbm.at[gidx_v.at[p]], rows_v.at[p],
                                  gsems[p]).wait()
            for r in range(R):
                for cc in range(ncc):
                    acc = rows_v[p, r * KP, pl.ds(cc * 16, 16)]
                    for j in range(1, KP):
                        acc = jnp.maximum(
                            acc, rows_v[p, r * KP + j, pl.ds(cc * 16, 16)]
                        )
                    out_v[p, r, pl.ds(cc * 16, 16)] = acc
            pltpu.async_copy(out_v.at[p], out_hbm.at[pl.ds(r0, R)], osems[p])

        fire(0, 0)

        def super_step(si, _):
            c0 = si * 2
            fire(1, c0 + 1)
            consume(0, c0)

            @pl.when(c0 + 2 < n_chunks)
            def _():
                fire(0, c0 + 2)

            consume(1, c0 + 1)
            pltpu.make_async_copy(out_v.at[0],
                                  out_hbm.at[pl.ds(base_row, R)], osems[0]
                                  ).wait()
            pltpu.make_async_copy(out_v.at[1],
                                  out_hbm.at[pl.ds(base_row, R)], osems[1]
                                  ).wait()
            return 0

        lax.fori_loop(0, n_chunks // 2, super_step, 0)

    return k(table, idx_flat)


def _head1_body(m_ref, v_ref, x1_ref, x2_ref, x3_ref, wf_ref, bf_ref, o_ref):
    x4 = _leaky(m_ref[0] + v_ref[0])
    xcat = jnp.concatenate(
        [x1_ref[0][:, :64], x2_ref[0][:, :64], x3_ref[0], x4], axis=1)
    y = _leaky(
        jnp.dot(xcat, wf_ref[...], preferred_element_type=jnp.float32)
        + bf_ref[...]
    )
    mx = jnp.max(y, axis=0, keepdims=True)
    mn = jnp.sum(y, axis=0, keepdims=True) * np.float32(1.0 / N)
    o_ref[0] = jnp.concatenate([mx, mn], axis=1)


def _head1(m4, v4, x1, x2, x3, wf_t, bfp):
    return pl.pallas_call(
        _head1_body,
        grid=(B,),
        in_specs=[
            pl.BlockSpec((1, N, 256), lambda b: (b, 0, 0)),
            pl.BlockSpec((1, N, 256), lambda b: (b, 0, 0)),
            pl.BlockSpec((1, N, 128), lambda b: (b, 0, 0)),
            pl.BlockSpec((1, N, 128), lambda b: (b, 0, 0)),
            pl.BlockSpec((1, N, 128), lambda b: (b, 0, 0)),
            pl.BlockSpec((512, 1024), lambda b: (0, 0)),
            pl.BlockSpec((1, 1024), lambda b: (0, 0)),
        ],
        out_specs=pl.BlockSpec((1, 1, 2048), lambda b: (b, 0, 0)),
        out_shape=jax.ShapeDtypeStruct((B, 1, 2048), jnp.float32),
    )(m4, v4, x1, x2, x3, wf_t, bfp)


def _head2_body(p_ref, w1_ref, b1_ref, w2_ref, b2_ref, wo_ref, bo_ref, o_ref):
    h = _leaky(
        jnp.dot(p_ref[...], w1_ref[...], preferred_element_type=jnp.float32)
        + b1_ref[...]
    )
    h = _leaky(
        jnp.dot(h, w2_ref[...], preferred_element_type=jnp.float32)
        + b2_ref[...]
    )
    o_ref[...] = (
        jnp.dot(h, wo_ref[...], preferred_element_type=jnp.float32)
        + bo_ref[...]
    )


def _head2(pooled, w1t, b1p, w2t, b2p, wot, bop):
    return pl.pallas_call(
        _head2_body,
        in_specs=[pl.BlockSpec(a.shape, lambda: tuple(0 for _ in a.shape))
                  for a in (pooled, w1t, b1p, w2t, b2p, wot, bop)],
        out_specs=pl.BlockSpec((B, 128), lambda: (0, 0)),
        out_shape=jax.ShapeDtypeStruct((B, 128), jnp.float32),
    )(pooled, w1t, b1p, w2t, b2p, wot, bop)


def _fold_conv(W, bvec, g, e, ci_pad, co_pad):
    Wp = W * g[:, None]
    bp = bvec * g + e
    C = W.shape[1] // 2
    co = W.shape[0]
    Wa = Wp[:, :C].T
    Wba = (Wp[:, C:] - Wp[:, :C]).T

    def pad2(a):
        return jnp.pad(a, ((0, ci_pad - C), (0, co_pad - co)))

    return pad2(Wa), pad2(Wba), jnp.pad(bp, (0, co_pad - co))[None, :]


def kernel(cloud, indices, W1, b1, g1, e1, W2, b2, g2, e2, W3, b3, g3, e3,
           W4, b4, g4, e4, Wf, bf, gf, ef, Wm1, bm1, gm1, em1,
           Wm2, bm2, gm2, em2, Wo, bo):
    wa1, wba1, bp1 = _fold_conv(W1, b1, g1, e1, 8, 128)
    wa2, wba2, bp2 = _fold_conv(W2, b2, g2, e2, 128, 128)
    wa3, wba3, bp3 = _fold_conv(W3, b3, g3, e3, 128, 128)
    wa4, wba4, bp4 = _fold_conv(W4, b4, g4, e4, 128, 256)
    xpad = jnp.concatenate(
        [cloud, jnp.zeros((B, N, 5), jnp.float32)], axis=-1)

    idx1 = indices.astype(jnp.int32)

    u1, v1 = _layer1(xpad, wa1, wba1, bp1, 128)
    m1 = _gather_max(u1.reshape(B * N, 128), idx1.reshape(-1))
    m1 = m1.reshape(B, N, 128)

    x1, idx2, u2, v2 = _layer(m1, v1, wa2, wba2, bp2)
    m2 = _gather_max(u2.reshape(B * N, 128), idx2.reshape(-1)).reshape(B, N, 128)

    x2, idx3, u3, v3 = _layer(m2, v2, wa3, wba3, bp3)
    m3 = _gather_max(u3.reshape(B * N, 128), idx3.reshape(-1)).reshape(B, N, 128)

    x3, idx4, u4, v4 = _layer(m3, v3, wa4, wba4, bp4)
    m4 = _gather_max(u4.reshape(B * N, 256), idx4.reshape(-1)).reshape(B, N, 256)

    wf_t = (Wf * gf[:, None]).T
    bfp = (bf * gf + ef)[None, :]
    pooled = _head1(m4, v4, x1, x2, x3, wf_t, bfp).reshape(B, 2048)

    w1t = (Wm1 * gm1[:, None]).T
    b1p = (bm1 * gm1 + em1)[None, :]
    w2t = (Wm2 * gm2[:, None]).T
    b2p = (bm2 * gm2 + em2)[None, :]
    wot = jnp.concatenate([Wo.T, jnp.zeros((256, 88), jnp.float32)], axis=1)
    bop = jnp.concatenate([bo, jnp.zeros((88,), jnp.float32)])[None, :]
    out = _head2(pooled, w1t, b1p, w2t, b2p, wot, bop)
    return out[:, :40]


# speedup vs baseline: 11.2353x; 1.3601x over previous
import functools

import jax
import jax.numpy as jnp
import numpy as np
from jax import lax
from jax.experimental import pallas as pl
from jax.experimental.pallas import tpu as pltpu
from jax.experimental.pallas import tpu_sc as plsc

B = 16
N = 1024
K = 20
KP = 20
NEG = np.float32(-np.inf)


def _leaky(y):
    return jnp.maximum(y, 0.2 * y)


def _layer1_body(x_ref, wa_ref, wba_ref, bp_ref, u_ref, v_ref):
    x = x_ref[0]
    u_ref[0] = jnp.dot(x, wa_ref[...], preferred_element_type=jnp.float32)
    v_ref[0] = (
        jnp.dot(x, wba_ref[...], preferred_element_type=jnp.float32)
        + bp_ref[...]
    )


def _layer1(xpad, wa, wba, bp, co):
    ci = xpad.shape[-1]
    nb = xpad.shape[0]
    return pl.pallas_call(
        _layer1_body,
        grid=(nb,),
        in_specs=[
            pl.BlockSpec((1, N, ci), lambda b: (b, 0, 0)),
            pl.BlockSpec((ci, co), lambda b: (0, 0)),
            pl.BlockSpec((ci, co), lambda b: (0, 0)),
            pl.BlockSpec((1, co), lambda b: (0, 0)),
        ],
        out_specs=[
            pl.BlockSpec((1, N, co), lambda b: (b, 0, 0)),
            pl.BlockSpec((1, N, co), lambda b: (b, 0, 0)),
        ],
        out_shape=[
            jax.ShapeDtypeStruct((nb, N, co), jnp.float32),
            jax.ShapeDtypeStruct((nb, N, co), jnp.float32),
        ],
    )(xpad, wa, wba, bp)


def _layer_body(m_ref, v_ref, wa_ref, wba_ref, bp_ref,
                x_ref, idx_ref, u_ref, vo_ref, d_ref):
    m = m_ref[0]
    v = v_ref[0]
    x = _leaky(m + v)
    x_ref[0] = x
    xx = jnp.sum(x * x, axis=1, keepdims=True)
    inner = lax.dot_general(
        x, x, (((1,), (1,)), ((), ())), preferred_element_type=jnp.float32
    )
    d_ref[...] = -((xx - 2.0 * inner) + jnp.reshape(xx, (1, N)))

    lane = lax.broadcasted_iota(jnp.int32, (N, N), 1)
    lane24 = lax.broadcasted_iota(jnp.int32, (N, KP), 1)
    n_i32 = np.int32(N)

    def step(j, carry):
        pv, pi, acc = carry
        d = d_ref[...]
        elig = (d < pv) | ((d == pv) & (lane > pi))
        rm = jnp.max(jnp.where(elig, d, NEG), axis=1, keepdims=True)
        pi_eff = jnp.where(rm == pv, pi, -1)
        msk = (d == rm) & (lane > pi_eff)
        ii = jnp.min(jnp.where(msk, lane, n_i32), axis=1, keepdims=True)
        acc = jnp.where(lane24 == j, ii, acc)
        return rm, ii, acc

    pv0 = jnp.full((N, 1), jnp.inf, jnp.float32)
    pi0 = jnp.full((N, 1), -1, jnp.int32)
    acc0 = jnp.zeros((N, KP), jnp.int32)
    _, _, acc = lax.fori_loop(0, K, step, (pv0, pi0, acc0))
    idx_ref[0] = acc

    u_ref[0] = jnp.dot(x, wa_ref[...], preferred_element_type=jnp.float32)
    vo_ref[0] = (
        jnp.dot(x, wba_ref[...], preferred_element_type=jnp.float32)
        + bp_ref[...]
    )


def _layer(m, v, wa, wba, bp):
    ci, co = wa.shape
    nb = m.shape[0]
    return pl.pallas_call(
        _layer_body,
        grid=(nb,),
        in_specs=[
            pl.BlockSpec((1, N, ci), lambda b: (b, 0, 0)),
            pl.BlockSpec((1, N, ci), lambda b: (b, 0, 0)),
            pl.BlockSpec((ci, co), lambda b: (0, 0)),
            pl.BlockSpec((ci, co), lambda b: (0, 0)),
            pl.BlockSpec((1, co), lambda b: (0, 0)),
        ],
        out_specs=[
            pl.BlockSpec((1, N, ci), lambda b: (b, 0, 0)),
            pl.BlockSpec((1, N, KP), lambda b: (b, 0, 0)),
            pl.BlockSpec((1, N, co), lambda b: (b, 0, 0)),
            pl.BlockSpec((1, N, co), lambda b: (b, 0, 0)),
        ],
        out_shape=[
            jax.ShapeDtypeStruct((nb, N, ci), jnp.float32),
            jax.ShapeDtypeStruct((nb, N, KP), jnp.int32),
            jax.ShapeDtypeStruct((nb, N, co), jnp.float32),
            jax.ShapeDtypeStruct((nb, N, co), jnp.float32),
        ],
        scratch_shapes=[pltpu.VMEM((N, N), jnp.float32)],
    )(m, v, wa, wba, bp)


def _gather_max(table, idx_flat):
    co = table.shape[-1]
    nrows = table.shape[0]
    nw = 32
    rows_per_w = nrows // nw
    wpb = N // rows_per_w
    R = 4
    n_chunks = rows_per_w // R
    ncc = co // 16

    mesh = plsc.VectorSubcoreMesh(core_axis_name="c", subcore_axis_name="s")

    @functools.partial(
        pl.kernel,
        out_type=jax.ShapeDtypeStruct((nrows, co), jnp.float32),
        mesh=mesh,
        scratch_types=[
            pltpu.VMEM((2, R * KP), jnp.int32),
            pltpu.VMEM((2, R * KP), jnp.int32),
            pltpu.VMEM((2, R * KP, co), jnp.float32),
            pltpu.VMEM((2, R, co), jnp.float32),
            pltpu.SemaphoreType.DMA,
            pltpu.SemaphoreType.DMA,
            pltpu.SemaphoreType.DMA,
            pltpu.SemaphoreType.DMA,
        ],
    )
    def k(table_hbm, idx_hbm, out_hbm, idxr_v, gidx_v, rows_v, out_v,
          gs0, gs1, os0, os1):
        wid = lax.axis_index("s") * 2 + lax.axis_index("c")
        base_row = wid * rows_per_w
        boff = (wid // wpb) * N
        gsems = (gs0, gs1)
        osems = (os0, os1)

        def fire(p, c):
            r0 = base_row + c * R
            pltpu.sync_copy(idx_hbm.at[pl.ds(r0 * KP, R * KP)],
                            idxr_v.at[p])
            for t in range(R * KP // 16):
                gidx_v[p, pl.ds(t * 16, 16)] = (
                    idxr_v[p, pl.ds(t * 16, 16)] + boff)
            pltpu.async_copy(table_hbm.at[gidx_v.at[p]], rows_v.at[p],
                             gsems[p])

        def consume(p, c):
            r0 = base_row + c * R
            pltpu.make_async_copy(table_hbm.at[gidx_v.at[p]], rows_v.at[p],
                                  gsems[p]).wait()
            for r in range(R):
                for cc in range(ncc):
                    acc = rows_v[p, r * KP, pl.ds(cc * 16, 16)]
                    for j in range(1, KP):
                        acc = jnp.maximum(
                            acc, rows_v[p, r * KP + j, pl.ds(cc * 16, 16)]
                        )
                    out_v[p, r, pl.ds(cc * 16, 16)] = acc
            pltpu.async_copy(out_v.at[p], out_hbm.at[pl.ds(r0, R)], osems[p])

        fire(0, 0)

        def super_step(si, _):
            c0 = si * 2
            fire(1, c0 + 1)
            consume(0, c0)

            @pl.when(c0 + 2 < n_chunks)
            def _():
                fire(0, c0 + 2)

            consume(1, c0 + 1)
            pltpu.make_async_copy(out_v.at[0],
                                  out_hbm.at[pl.ds(base_row, R)], osems[0]
                                  ).wait()
            pltpu.make_async_copy(out_v.at[1],
                                  out_hbm.at[pl.ds(base_row, R)], osems[1]
                                  ).wait()
            return 0

        lax.fori_loop(0, n_chunks // 2, super_step, 0)

    return k(table, idx_flat)


def _head1_body(m_ref, v_ref, x1_ref, x2_ref, x3_ref, wf_ref, bf_ref, o_ref):
    x4 = _leaky(m_ref[0] + v_ref[0])
    xcat = jnp.concatenate(
        [x1_ref[0][:, :64], x2_ref[0][:, :64], x3_ref[0], x4], axis=1)
    y = _leaky(
        jnp.dot(xcat, wf_ref[...], preferred_element_type=jnp.float32)
        + bf_ref[...]
    )
    mx = jnp.max(y, axis=0, keepdims=True)
    mn = jnp.sum(y, axis=0, keepdims=True) * np.float32(1.0 / N)
    o_ref[0] = jnp.concatenate([mx, mn], axis=1)


def _head1(m4, v4, x1, x2, x3, wf_t, bfp):
    nb = m4.shape[0]
    return pl.pallas_call(
        _head1_body,
        grid=(nb,),
        in_specs=[
            pl.BlockSpec((1, N, 256), lambda b: (b, 0, 0)),
            pl.BlockSpec((1, N, 256), lambda b: (b, 0, 0)),
            pl.BlockSpec((1, N, 128), lambda b: (b, 0, 0)),
            pl.BlockSpec((1, N, 128), lambda b: (b, 0, 0)),
            pl.BlockSpec((1, N, 128), lambda b: (b, 0, 0)),
            pl.BlockSpec((512, 1024), lambda b: (0, 0)),
            pl.BlockSpec((1, 1024), lambda b: (0, 0)),
        ],
        out_specs=pl.BlockSpec((1, 1, 2048), lambda b: (b, 0, 0)),
        out_shape=jax.ShapeDtypeStruct((nb, 1, 2048), jnp.float32),
    )(m4, v4, x1, x2, x3, wf_t, bfp)


def _head2_body(p_ref, w1_ref, b1_ref, w2_ref, b2_ref, wo_ref, bo_ref, o_ref):
    h = _leaky(
        jnp.dot(p_ref[...], w1_ref[...], preferred_element_type=jnp.float32)
        + b1_ref[...]
    )
    h = _leaky(
        jnp.dot(h, w2_ref[...], preferred_element_type=jnp.float32)
        + b2_ref[...]
    )
    o_ref[...] = (
        jnp.dot(h, wo_ref[...], preferred_element_type=jnp.float32)
        + bo_ref[...]
    )


def _head2(pooled, w1t, b1p, w2t, b2p, wot, bop):
    return pl.pallas_call(
        _head2_body,
        in_specs=[pl.BlockSpec(a.shape, lambda: tuple(0 for _ in a.shape))
                  for a in (pooled, w1t, b1p, w2t, b2p, wot, bop)],
        out_specs=pl.BlockSpec((B, 128), lambda: (0, 0)),
        out_shape=jax.ShapeDtypeStruct((B, 128), jnp.float32),
    )(pooled, w1t, b1p, w2t, b2p, wot, bop)


def _fold_conv(W, bvec, g, e, ci_pad, co_pad):
    Wp = W * g[:, None]
    bp = bvec * g + e
    C = W.shape[1] // 2
    co = W.shape[0]
    Wa = Wp[:, :C].T
    Wba = (Wp[:, C:] - Wp[:, :C]).T

    def pad2(a):
        return jnp.pad(a, ((0, ci_pad - C), (0, co_pad - co)))

    return pad2(Wa), pad2(Wba), jnp.pad(bp, (0, co_pad - co))[None, :]


def kernel(cloud, indices, W1, b1, g1, e1, W2, b2, g2, e2, W3, b3, g3, e3,
           W4, b4, g4, e4, Wf, bf, gf, ef, Wm1, bm1, gm1, em1,
           Wm2, bm2, gm2, em2, Wo, bo):
    wa1, wba1, bp1 = _fold_conv(W1, b1, g1, e1, 8, 128)
    wa2, wba2, bp2 = _fold_conv(W2, b2, g2, e2, 128, 128)
    wa3, wba3, bp3 = _fold_conv(W3, b3, g3, e3, 128, 128)
    wa4, wba4, bp4 = _fold_conv(W4, b4, g4, e4, 128, 256)
    xpad = jnp.concatenate(
        [cloud, jnp.zeros((B, N, 5), jnp.float32)], axis=-1)

    idx1 = indices.astype(jnp.int32)
    wf_t = (Wf * gf[:, None]).T
    bfp = (bf * gf + ef)[None, :]

    HB = B // 2
    pooled_halves = []
    for h in range(2):
        sl = slice(h * HB, (h + 1) * HB)
        xp = xpad[sl]
        u1, v1 = _layer1(xp, wa1, wba1, bp1, 128)
        m1 = _gather_max(u1.reshape(HB * N, 128), idx1[sl].reshape(-1))
        m1 = m1.reshape(HB, N, 128)

        x1, idx2, u2, v2 = _layer(m1, v1, wa2, wba2, bp2)
        m2 = _gather_max(u2.reshape(HB * N, 128),
                         idx2.reshape(-1)).reshape(HB, N, 128)

        x2, idx3, u3, v3 = _layer(m2, v2, wa3, wba3, bp3)
        m3 = _gather_max(u3.reshape(HB * N, 128),
                         idx3.reshape(-1)).reshape(HB, N, 128)

        x3, idx4, u4, v4 = _layer(m3, v3, wa4, wba4, bp4)
        m4 = _gather_max(u4.reshape(HB * N, 256),
                         idx4.reshape(-1)).reshape(HB, N, 256)

        pooled_halves.append(
            _head1(m4, v4, x1, x2, x3, wf_t, bfp).reshape(HB, 2048))
    pooled = jnp.concatenate(pooled_halves, axis=0)

    w1t = (Wm1 * gm1[:, None]).T
    b1p = (bm1 * gm1 + em1)[None, :]
    w2t = (Wm2 * gm2[:, None]).T
    b2p = (bm2 * gm2 + em2)[None, :]
    wot = jnp.concatenate([Wo.T, jnp.zeros((256, 88), jnp.float32)], axis=1)
    bop = jnp.concatenate([bo, jnp.zeros((88,), jnp.float32)])[None, :]
    out = _head2(pooled, w1t, b1p, w2t, b2p, wot, bop)
    return out[:, :40]


# speedup vs baseline: 11.6436x; 1.0363x over previous
import functools

import jax
import jax.numpy as jnp
import numpy as np
from jax import lax
from jax.experimental import pallas as pl
from jax.experimental.pallas import tpu as pltpu
from jax.experimental.pallas import tpu_sc as plsc

B = 16
N = 1024
K = 20
KP = 20
NEG = np.float32(-np.inf)


def _leaky(y):
    return jnp.maximum(y, 0.2 * y)


def _layer1_body(x_ref, idx_ref, wa_ref, wba_ref, bp_ref,
                 u_ref, v_ref, idxg_ref):
    x = x_ref[0]
    u_ref[0] = jnp.dot(x, wa_ref[...], preferred_element_type=jnp.float32)
    v_ref[0] = (
        jnp.dot(x, wba_ref[...], preferred_element_type=jnp.float32)
        + bp_ref[...]
    )
    idxg_ref[0] = idx_ref[0] + pl.program_id(0) * N


def _layer1(xpad, idx1, wa, wba, bp, co):
    ci = xpad.shape[-1]
    nb = xpad.shape[0]
    return pl.pallas_call(
        _layer1_body,
        grid=(nb,),
        in_specs=[
            pl.BlockSpec((1, N, ci), lambda b: (b, 0, 0)),
            pl.BlockSpec((1, N, KP), lambda b: (b, 0, 0)),
            pl.BlockSpec((ci, co), lambda b: (0, 0)),
            pl.BlockSpec((ci, co), lambda b: (0, 0)),
            pl.BlockSpec((1, co), lambda b: (0, 0)),
        ],
        out_specs=[
            pl.BlockSpec((1, N, co), lambda b: (b, 0, 0)),
            pl.BlockSpec((1, N, co), lambda b: (b, 0, 0)),
            pl.BlockSpec((1, N, KP), lambda b: (b, 0, 0)),
        ],
        out_shape=[
            jax.ShapeDtypeStruct((nb, N, co), jnp.float32),
            jax.ShapeDtypeStruct((nb, N, co), jnp.float32),
            jax.ShapeDtypeStruct((nb, N, KP), jnp.int32),
        ],
    )(xpad, idx1, wa, wba, bp)


def _layer_body(m_ref, v_ref, wa_ref, wba_ref, bp_ref,
                x_ref, idx_ref, u_ref, vo_ref, d_ref):
    m = m_ref[0]
    v = v_ref[0]
    x = _leaky(m + v)
    x_ref[0] = x
    xx = jnp.sum(x * x, axis=1, keepdims=True)
    inner = lax.dot_general(
        x, x, (((1,), (1,)), ((), ())), preferred_element_type=jnp.float32
    )
    d_ref[...] = -((xx - 2.0 * inner) + jnp.reshape(xx, (1, N)))

    lane = lax.broadcasted_iota(jnp.int32, (N, N), 1)
    lane24 = lax.broadcasted_iota(jnp.int32, (N, KP), 1)
    n_i32 = np.int32(N)

    def step(j, carry):
        pv, pi, acc = carry
        d = d_ref[...]
        elig = (d < pv) | ((d == pv) & (lane > pi))
        rm = jnp.max(jnp.where(elig, d, NEG), axis=1, keepdims=True)
        pi_eff = jnp.where(rm == pv, pi, -1)
        msk = (d == rm) & (lane > pi_eff)
        ii = jnp.min(jnp.where(msk, lane, n_i32), axis=1, keepdims=True)
        acc = jnp.where(lane24 == j, ii, acc)
        return rm, ii, acc

    pv0 = jnp.full((N, 1), jnp.inf, jnp.float32)
    pi0 = jnp.full((N, 1), -1, jnp.int32)
    acc0 = jnp.zeros((N, KP), jnp.int32)
    _, _, acc = lax.fori_loop(0, K, step, (pv0, pi0, acc0))
    idx_ref[0] = acc + pl.program_id(0) * N

    u_ref[0] = jnp.dot(x, wa_ref[...], preferred_element_type=jnp.float32)
    vo_ref[0] = (
        jnp.dot(x, wba_ref[...], preferred_element_type=jnp.float32)
        + bp_ref[...]
    )


def _layer(m, v, wa, wba, bp):
    ci, co = wa.shape
    nb = m.shape[0]
    return pl.pallas_call(
        _layer_body,
        grid=(nb,),
        in_specs=[
            pl.BlockSpec((1, N, ci), lambda b: (b, 0, 0)),
            pl.BlockSpec((1, N, ci), lambda b: (b, 0, 0)),
            pl.BlockSpec((ci, co), lambda b: (0, 0)),
            pl.BlockSpec((ci, co), lambda b: (0, 0)),
            pl.BlockSpec((1, co), lambda b: (0, 0)),
        ],
        out_specs=[
            pl.BlockSpec((1, N, ci), lambda b: (b, 0, 0)),
            pl.BlockSpec((1, N, KP), lambda b: (b, 0, 0)),
            pl.BlockSpec((1, N, co), lambda b: (b, 0, 0)),
            pl.BlockSpec((1, N, co), lambda b: (b, 0, 0)),
        ],
        out_shape=[
            jax.ShapeDtypeStruct((nb, N, ci), jnp.float32),
            jax.ShapeDtypeStruct((nb, N, KP), jnp.int32),
            jax.ShapeDtypeStruct((nb, N, co), jnp.float32),
            jax.ShapeDtypeStruct((nb, N, co), jnp.float32),
        ],
        scratch_shapes=[pltpu.VMEM((N, N), jnp.float32)],
    )(m, v, wa, wba, bp)


def _gather_max(table, idx2d):
    co = table.shape[-1]
    nrows = table.shape[0]
    nw = 32
    rows_per_w = nrows // nw
    R = 4
    n_chunks = rows_per_w // R
    ncc = co // 16

    mesh = plsc.VectorSubcoreMesh(core_axis_name="c", subcore_axis_name="s")

    @functools.partial(
        pl.kernel,
        out_type=jax.ShapeDtypeStruct((nrows, co), jnp.float32),
        mesh=mesh,
        scratch_types=[
            pltpu.VMEM((n_chunks, R * KP), jnp.int32),
            pltpu.VMEM((2, R * KP, co), jnp.float32),
            pltpu.VMEM((2, R, co), jnp.float32),
            pltpu.SemaphoreType.DMA,
            pltpu.SemaphoreType.DMA,
            pltpu.SemaphoreType.DMA,
            pltpu.SemaphoreType.DMA,
        ],
    )
    def k(table_hbm, idx_hbm, out_hbm, idx_v, rows_v, out_v,
          gs0, gs1, os0, os1):
        wid = lax.axis_index("s") * 2 + lax.axis_index("c")
        base_row = wid * rows_per_w
        gsems = (gs0, gs1)
        osems = (os0, os1)

        pltpu.sync_copy(idx_hbm.at[pl.ds(wid * n_chunks, n_chunks)], idx_v)

        def fire(p, c):
            pltpu.async_copy(table_hbm.at[idx_v.at[c]], rows_v.at[p],
                             gsems[p])

        def consume(p, c):
            r0 = base_row + c * R
            pltpu.make_async_copy(table_hbm.at[idx_v.at[c]], rows_v.at[p],
                                  gsems[p]).wait()
            for r in range(R):
                for cc in range(ncc):
                    acc = rows_v[p, r * KP, pl.ds(cc * 16, 16)]
                    for j in range(1, KP):
                        acc = jnp.maximum(
                            acc, rows_v[p, r * KP + j, pl.ds(cc * 16, 16)]
                        )
                    out_v[p, r, pl.ds(cc * 16, 16)] = acc
            pltpu.async_copy(out_v.at[p], out_hbm.at[pl.ds(r0, R)], osems[p])

        fire(0, 0)

        def super_step(si, _):
            c0 = si * 2
            fire(1, c0 + 1)
            consume(0, c0)

            @pl.when(c0 + 2 < n_chunks)
            def _():
                fire(0, c0 + 2)

            consume(1, c0 + 1)
            pltpu.make_async_copy(out_v.at[0],
                                  out_hbm.at[pl.ds(base_row, R)], osems[0]
                                  ).wait()
            pltpu.make_async_copy(out_v.at[1],
                                  out_hbm.at[pl.ds(base_row, R)], osems[1]
                                  ).wait()
            return 0

        lax.fori_loop(0, n_chunks // 2, super_step, 0)

    return k(table, idx2d)


def _head1_body(m_ref, v_ref, x1_ref, x2_ref, x3_ref, wf_ref, bf_ref, o_ref):
    x4 = _leaky(m_ref[0] + v_ref[0])
    xcat = jnp.concatenate(
        [x1_ref[0][:, :64], x2_ref[0][:, :64], x3_ref[0], x4], axis=1)
    y = _leaky(
        jnp.dot(xcat, wf_ref[...], preferred_element_type=jnp.float32)
        + bf_ref[...]
    )
    mx = jnp.max(y, axis=0, keepdims=True)
    mn = jnp.sum(y, axis=0, keepdims=True) * np.float32(1.0 / N)
    o_ref[0] = jnp.concatenate([mx, mn], axis=1)


def _head1(m4, v4, x1, x2, x3, wf_t, bfp):
    nb = m4.shape[0]
    return pl.pallas_call(
        _head1_body,
        grid=(nb,),
        in_specs=[
            pl.BlockSpec((1, N, 256), lambda b: (b, 0, 0)),
            pl.BlockSpec((1, N, 256), lambda b: (b, 0, 0)),
            pl.BlockSpec((1, N, 128), lambda b: (b, 0, 0)),
            pl.BlockSpec((1, N, 128), lambda b: (b, 0, 0)),
            pl.BlockSpec((1, N, 128), lambda b: (b, 0, 0)),
            pl.BlockSpec((512, 1024), lambda b: (0, 0)),
            pl.BlockSpec((1, 1024), lambda b: (0, 0)),
        ],
        out_specs=pl.BlockSpec((1, 1, 2048), lambda b: (b, 0, 0)),
        out_shape=jax.ShapeDtypeStruct((nb, 1, 2048), jnp.float32),
    )(m4, v4, x1, x2, x3, wf_t, bfp)


def _head2_body(p_ref, w1_ref, b1_ref, w2_ref, b2_ref, wo_ref, bo_ref, o_ref):
    h = _leaky(
        jnp.dot(p_ref[...], w1_ref[...], preferred_element_type=jnp.float32)
        + b1_ref[...]
    )
    h = _leaky(
        jnp.dot(h, w2_ref[...], preferred_element_type=jnp.float32)
        + b2_ref[...]
    )
    o_ref[...] = (
        jnp.dot(h, wo_ref[...], preferred_element_type=jnp.float32)
        + bo_ref[...]
    )


def _head2(pooled, w1t, b1p, w2t, b2p, wot, bop):
    return pl.pallas_call(
        _head2_body,
        in_specs=[pl.BlockSpec(a.shape, lambda: tuple(0 for _ in a.shape))
                  for a in (pooled, w1t, b1p, w2t, b2p, wot, bop)],
        out_specs=pl.BlockSpec((B, 128), lambda: (0, 0)),
        out_shape=jax.ShapeDtypeStruct((B, 128), jnp.float32),
    )(pooled, w1t, b1p, w2t, b2p, wot, bop)


def _fold_conv(W, bvec, g, e, ci_pad, co_pad):
    Wp = W * g[:, None]
    bp = bvec * g + e
    C = W.shape[1] // 2
    co = W.shape[0]
    Wa = Wp[:, :C].T
    Wba = (Wp[:, C:] - Wp[:, :C]).T

    def pad2(a):
        return jnp.pad(a, ((0, ci_pad - C), (0, co_pad - co)))

    return pad2(Wa), pad2(Wba), jnp.pad(bp, (0, co_pad - co))[None, :]


def kernel(cloud, indices, W1, b1, g1, e1, W2, b2, g2, e2, W3, b3, g3, e3,
           W4, b4, g4, e4, Wf, bf, gf, ef, Wm1, bm1, gm1, em1,
           Wm2, bm2, gm2, em2, Wo, bo):
    wa1, wba1, bp1 = _fold_conv(W1, b1, g1, e1, 8, 128)
    wa2, wba2, bp2 = _fold_conv(W2, b2, g2, e2, 128, 128)
    wa3, wba3, bp3 = _fold_conv(W3, b3, g3, e3, 128, 128)
    wa4, wba4, bp4 = _fold_conv(W4, b4, g4, e4, 128, 256)
    xpad = jnp.concatenate(
        [cloud, jnp.zeros((B, N, 5), jnp.float32)], axis=-1)

    idx1 = indices.astype(jnp.int32)
    wf_t = (Wf * gf[:, None]).T
    bfp = (bf * gf + ef)[None, :]

    HB = B // 2
    pooled_halves = []
    for h in range(2):
        sl = slice(h * HB, (h + 1) * HB)
        nix = HB * N // 4
        xp = xpad[sl]
        u1, v1, idxg1 = _layer1(xp, idx1[sl], wa1, wba1, bp1, 128)
        m1 = _gather_max(u1.reshape(HB * N, 128), idxg1.reshape(nix, -1))
        m1 = m1.reshape(HB, N, 128)

        x1, idx2, u2, v2 = _layer(m1, v1, wa2, wba2, bp2)
        m2 = _gather_max(u2.reshape(HB * N, 128),
                         idx2.reshape(nix, -1)).reshape(HB, N, 128)

        x2, idx3, u3, v3 = _layer(m2, v2, wa3, wba3, bp3)
        m3 = _gather_max(u3.reshape(HB * N, 128),
                         idx3.reshape(nix, -1)).reshape(HB, N, 128)

        x3, idx4, u4, v4 = _layer(m3, v3, wa4, wba4, bp4)
        m4 = _gather_max(u4.reshape(HB * N, 256),
                         idx4.reshape(nix, -1)).reshape(HB, N, 256)

        pooled_halves.append(
            _head1(m4, v4, x1, x2, x3, wf_t, bfp).reshape(HB, 2048))
    pooled = jnp.concatenate(pooled_halves, axis=0)

    w1t = (Wm1 * gm1[:, None]).T
    b1p = (bm1 * gm1 + em1)[None, :]
    w2t = (Wm2 * gm2[:, None]).T
    b2p = (bm2 * gm2 + em2)[None, :]
    wot = jnp.concatenate([Wo.T, jnp.zeros((256, 88), jnp.float32)], axis=1)
    bop = jnp.concatenate([bo, jnp.zeros((88,), jnp.float32)])[None, :]
    out = _head2(pooled, w1t, b1p, w2t, b2p, wot, bop)
    return out[:, :40]
